# baseline (speedup 1.0000x reference)
import functools

import jax
import jax.numpy as jnp
from jax import lax
from jax.experimental import pallas as pl
from jax.experimental.pallas import tpu as pltpu
from jax.experimental.pallas import tpu_sc as plsc

D_MODEL = 128
NUM_CORES = 2
NUM_SUBCORES = 16
NUM_WORKERS = NUM_CORES * NUM_SUBCORES

IDX_ROW = 128
K = 1
NBUF = 3


def _make_gather(n_rows: int):
    rows_per_w = n_rows // NUM_WORKERS
    n_chunks = rows_per_w // K
    assert n_chunks > 2 * NBUF
    mesh = plsc.VectorSubcoreMesh(
        core_axis_name="c", subcore_axis_name="s", num_cores=NUM_CORES
    )

    @functools.partial(
        pl.kernel,
        out_type=jax.ShapeDtypeStruct((n_rows, IDX_ROW, D_MODEL), jnp.float32),
        mesh=mesh,
        scratch_types=[
            pltpu.VMEM((NBUF, K, IDX_ROW), jnp.int32),
            pltpu.VMEM((NBUF, K, IDX_ROW, D_MODEL), jnp.float32),
            pltpu.VMEM_SHARED((8192, D_MODEL), jnp.float32),
            pltpu.SemaphoreType.DMA,
            pltpu.SemaphoreType.DMA,
            pltpu.SemaphoreType.DMA,
        ],
    )
    def gather_kernel(idx_hbm, pe_hbm, out_hbm, idx_v, rows_v, pe_sh, isem, gsem, osem):
        sid = lax.axis_index("s")
        @pl.when(sid == 0)
        def _stage():
            pltpu.sync_copy(pe_hbm, pe_sh)

        plsc.subcore_barrier()
        wid = lax.axis_index("s") * NUM_CORES + lax.axis_index("c")
        base = wid * rows_per_w
        last_row = n_rows - K

        def fire_idx(g, b):
            r = jnp.minimum(base + g * K, last_row)
            pltpu.async_copy(idx_hbm.at[pl.ds(r, K)], idx_v.at[b], isem)

        def wait_idx(b):
            pltpu.make_async_copy(
                idx_hbm.at[pl.ds(0, K)], idx_v.at[b], isem
            ).wait()

        def wait_out(b):
            pltpu.make_async_copy(
                rows_v.at[b], out_hbm.at[pl.ds(0, K)], osem
            ).wait()

        def fire_gathers(b):
            for j in range(K):
                pltpu.async_copy(pe_sh.at[idx_v.at[b, j]], rows_v.at[b, j], gsem)

        def drain_gathers(b):
            for j in range(K):
                pltpu.make_async_copy(
                    pe_sh.at[idx_v.at[b, j]], rows_v.at[b, j], gsem
                ).wait()

        def fire_out(g, b):
            pltpu.async_copy(rows_v.at[b], out_hbm.at[pl.ds(base + g * K, K)], osem)

        def step(g, b, f, first_out, fire_ahead):
            drain_gathers(b)
            fire_out(g, b)
            fire_idx(g + NBUF, b)
            if fire_ahead:
                wait_idx(f)
                if not first_out:
                    wait_out(f)
                fire_gathers(f)

        for b in range(NBUF):
            fire_idx(b, b)
        for b in range(2):
            wait_idx(b)
            fire_gathers(b)
        step(0, 0, 2, first_out=True, fire_ahead=True)

        def ring(g, carry):
            b = lax.rem(g, NBUF)
            f = lax.rem(g + 2, NBUF)
            step(g, b, f, first_out=False, fire_ahead=True)
            return carry

        lax.fori_loop(1, n_chunks - 2, ring, 0)
        for g in range(n_chunks - 2, n_chunks):
            step(g, g % NBUF, (g + 2) % NBUF, first_out=False, fire_ahead=False)

        for b in range(NBUF):
            wait_out(b)
            wait_idx(b)

    return gather_kernel


def kernel(edge_type, pe):
    batch, hist = edge_type.shape
    total = batch * hist
    n_rows = total // IDX_ROW
    idx2d = edge_type.reshape(n_rows, IDX_ROW)
    out = _make_gather(n_rows)(idx2d, pe)
    return out.reshape(batch, hist, D_MODEL)
